# baseline (speedup 1.0000x reference)
import functools

import jax
import jax.numpy as jnp
from jax import lax
from jax.experimental import pallas as pl

B, N, D, F, K = 64, 512, 768, 384, 40
BB = 8
NSPLIT = 8
NH = N // NSPLIT
KSPLIT = 256


def _dot(a, b):
    return jnp.dot(a, b, preferred_element_type=jnp.float32)


def _layer1(xh, wh, b1):
    acc = _dot(xh[:, :KSPLIT], wh[:KSPLIT, :])
    for k0 in range(KSPLIT, D, KSPLIT):
        acc = acc + _dot(xh[:, k0 : k0 + KSPLIT], wh[k0 : k0 + KSPLIT, :])
    return jnp.maximum(acc + b1, 0.0)


def _body(*refs):
    x_refs = refs[:NSPLIT]
    w1_ref, b1_ref, w2_ref, out_ref = refs[NSPLIT:]
    wh = w1_ref[...].astype(jnp.bfloat16)
    xhs = [r[...].reshape(BB * NH, D).astype(jnp.bfloat16) for r in x_refs]
    hs = [_layer1(xq, wh, b1_ref[...]).astype(jnp.bfloat16) for xq in xhs]
    w2c = w2_ref[...].astype(jnp.bfloat16)

    x3s = [xq.reshape(BB, NH, D) for xq in xhs]
    for i in range(BB):
        s_col = jnp.concatenate(
            [_dot(hq[i * NH : (i + 1) * NH, :], w2c) for hq in hs], axis=0
        )
        s_row = s_col.T

        im = lax.broadcasted_iota(jnp.int32, (N, N), 0)
        inn = lax.broadcasted_iota(jnp.int32, (N, N), 1)
        beats = (s_col > s_row) | ((s_col == s_row) & (im < inn))
        rank = jnp.sum(beats.astype(jnp.int32), axis=0, keepdims=True)

        jk = lax.broadcasted_iota(jnp.int32, (K, N), 0)
        p = (rank == jk).astype(jnp.bfloat16)

        acc = _dot(p[:, :NH], x3s[0][i])
        for q in range(1, NSPLIT):
            acc = acc + _dot(p[:, q * NH : (q + 1) * NH], x3s[q][i])
        out_ref[i, :, :] = acc


@jax.jit
def _run(patch_features, W1, b1, W2):
    grid = (B // BB,)
    x_specs = [
        pl.BlockSpec((BB, NH, D), functools.partial(lambda q, i: (i, q, 0), q))
        for q in range(NSPLIT)
    ]
    return pl.pallas_call(
        _body,
        grid=grid,
        in_specs=x_specs
        + [
            pl.BlockSpec((D, F), lambda i: (0, 0)),
            pl.BlockSpec((1, F), lambda i: (0, 0)),
            pl.BlockSpec((F, 1), lambda i: (0, 0)),
        ],
        out_specs=pl.BlockSpec((BB, K, D), lambda i: (i, 0, 0)),
        out_shape=jax.ShapeDtypeStruct((B, K, D), jnp.float32),
    )(*([patch_features] * NSPLIT), W1, b1, W2)


def kernel(patch_features, W1, b1, W2, b2):
    del b2
    b1r = b1.reshape(1, F)
    return _run(patch_features, W1, b1r, W2)


# speedup vs baseline: 1.0955x; 1.0955x over previous
import functools

import jax
import jax.numpy as jnp
from jax import lax
from jax.experimental import pallas as pl

B, N, D, F, K = 64, 512, 768, 384, 40
BB = 8
NSPLIT = 4
NH = N // NSPLIT
KSPLIT = 256


def _dot(a, b):
    return jnp.dot(a, b, preferred_element_type=jnp.float32)


def _layer1(xh, wh, b1):
    acc = _dot(xh[:, :KSPLIT], wh[:KSPLIT, :])
    for k0 in range(KSPLIT, D, KSPLIT):
        acc = acc + _dot(xh[:, k0 : k0 + KSPLIT], wh[k0 : k0 + KSPLIT, :])
    return jnp.maximum(acc + b1, 0.0)


def _body(*refs):
    x_refs = refs[:NSPLIT]
    w1_ref, b1_ref, w2_ref, out_ref = refs[NSPLIT:]
    wh = w1_ref[...].astype(jnp.bfloat16)
    xhs = [r[...].reshape(BB * NH, D).astype(jnp.bfloat16) for r in x_refs]
    hs = [_layer1(xq, wh, b1_ref[...]).astype(jnp.bfloat16) for xq in xhs]
    w2c = w2_ref[...].astype(jnp.bfloat16)

    x3s = [xq.reshape(BB, NH, D) for xq in xhs]
    for i in range(BB):
        s_col = jnp.concatenate(
            [_dot(hq[i * NH : (i + 1) * NH, :], w2c) for hq in hs], axis=0
        )
        s_row = s_col.T

        im = lax.broadcasted_iota(jnp.int32, (N, N), 0)
        inn = lax.broadcasted_iota(jnp.int32, (N, N), 1)
        beats = (s_col > s_row) | ((s_col == s_row) & (im < inn))
        rank = jnp.sum(beats.astype(jnp.int32), axis=0, keepdims=True)

        jk = lax.broadcasted_iota(jnp.int32, (K, N), 0)
        p = (rank == jk).astype(jnp.bfloat16)

        acc = _dot(p[:, :NH], x3s[0][i])
        for q in range(1, NSPLIT):
            acc = acc + _dot(p[:, q * NH : (q + 1) * NH], x3s[q][i])
        out_ref[i, :, :] = acc


@jax.jit
def _run(patch_features, W1, b1, W2):
    grid = (B // BB,)
    x_specs = [
        pl.BlockSpec((BB, NH, D), functools.partial(lambda q, i: (i, q, 0), q))
        for q in range(NSPLIT)
    ]
    return pl.pallas_call(
        _body,
        grid=grid,
        in_specs=x_specs
        + [
            pl.BlockSpec((D, F), lambda i: (0, 0)),
            pl.BlockSpec((1, F), lambda i: (0, 0)),
            pl.BlockSpec((F, 1), lambda i: (0, 0)),
        ],
        out_specs=pl.BlockSpec((BB, K, D), lambda i: (i, 0, 0)),
        out_shape=jax.ShapeDtypeStruct((B, K, D), jnp.float32),
    )(*([patch_features] * NSPLIT), W1, b1, W2)


def kernel(patch_features, W1, b1, W2, b2):
    del b2
    b1r = b1.reshape(1, F)
    return _run(patch_features, W1, b1r, W2)
